# Initial kernel scaffold; baseline (speedup 1.0000x reference)
import jax
import jax.numpy as jnp
from jax.experimental import pallas as pl


def kernel(x, router_weight, w1, w2, w3):
    raise NotImplementedError("write your pallas kernel here")


# speedup vs baseline: 1.0737x; 1.0737x over previous
import jax
import jax.numpy as jnp
from jax.experimental import pallas as pl
from jax.experimental.pallas import tpu as pltpu

D_MODEL = 1024
D_FF = 4096
N_EXPERTS = 8
TM = 1024
TF = 1024
J = D_FF // TF


def _moe_dense_kernel(xb_ref, rw_ref, w1_ref, w2_ref, w3_ref, out_ref,
                      coef_ref, acc_ref):
    e = pl.program_id(1)
    j = pl.program_id(2)

    @pl.when((e == 0) & (j == 0))
    def _router():
        logits = jnp.dot(xb_ref[...], rw_ref[...],
                         preferred_element_type=jnp.float32)
        mx = jnp.max(logits, axis=-1, keepdims=True)
        ex = jnp.exp(logits - mx)
        p = ex / jnp.sum(ex, axis=-1, keepdims=True)
        lane = jax.lax.broadcasted_iota(jnp.int32, p.shape, 1)
        m1 = jnp.max(p, axis=-1, keepdims=True)
        i1 = jnp.min(jnp.where(p == m1, lane, N_EXPERTS), axis=-1,
                     keepdims=True)
        mask1 = lane == i1
        pm = jnp.where(mask1, -1.0, p)
        m2 = jnp.max(pm, axis=-1, keepdims=True)
        i2 = jnp.min(jnp.where(pm == m2, lane, N_EXPERTS), axis=-1,
                     keepdims=True)
        mask2 = lane == i2
        coef_ref[...] = jnp.where(mask1 | mask2, p, 0.0) / (m1 + m2)
        acc_ref[...] = jnp.zeros_like(acc_ref)

    xb = xb_ref[...]
    gate = jnp.dot(xb, w1_ref[0], preferred_element_type=jnp.float32)
    val = jnp.dot(xb, w2_ref[0], preferred_element_type=jnp.float32)
    h = (gate * jax.nn.sigmoid(gate) * val).astype(jnp.bfloat16)
    eo = jnp.dot(h, w3_ref[0], preferred_element_type=jnp.float32)
    lane = jax.lax.broadcasted_iota(jnp.int32, coef_ref.shape, 1)
    c = jnp.sum(jnp.where(lane == e, coef_ref[...], 0.0), axis=-1,
                keepdims=True)
    acc_ref[...] += eo * c

    @pl.when((e == N_EXPERTS - 1) & (j == J - 1))
    def _finish():
        out_ref[...] = acc_ref[...]


def kernel(x, router_weight, w1, w2, w3):
    batch, seq, d = x.shape
    xb = x.reshape(seq, d).astype(jnp.bfloat16)
    rwb = router_weight.astype(jnp.bfloat16)
    w1b = w1.astype(jnp.bfloat16)
    w2b = w2.astype(jnp.bfloat16)
    w3b = w3.astype(jnp.bfloat16)
    mt = seq // TM
    out = pl.pallas_call(
        _moe_dense_kernel,
        grid=(mt, N_EXPERTS, J),
        in_specs=[
            pl.BlockSpec((TM, D_MODEL), lambda m, e, j: (m, 0)),
            pl.BlockSpec((D_MODEL, N_EXPERTS), lambda m, e, j: (0, 0)),
            pl.BlockSpec((1, D_MODEL, TF), lambda m, e, j: (e, 0, j)),
            pl.BlockSpec((1, D_MODEL, TF), lambda m, e, j: (e, 0, j)),
            pl.BlockSpec((1, TF, D_MODEL), lambda m, e, j: (e, j, 0)),
        ],
        out_specs=pl.BlockSpec((TM, D_MODEL), lambda m, e, j: (m, 0)),
        out_shape=jax.ShapeDtypeStruct((seq, d), jnp.float32),
        scratch_shapes=[
            pltpu.VMEM((TM, N_EXPERTS), jnp.float32),
            pltpu.VMEM((TM, D_MODEL), jnp.float32),
        ],
        compiler_params=pltpu.CompilerParams(
            dimension_semantics=("parallel", "arbitrary", "arbitrary"),
        ),
    )(xb, rwb, w1b, w2b, w3b)
    return out.reshape(batch, seq, d)


# speedup vs baseline: 1.2779x; 1.1902x over previous
import jax
import jax.numpy as jnp
from jax.experimental import pallas as pl
from jax.experimental.pallas import tpu as pltpu

D_MODEL = 1024
D_FF = 4096
N_EXPERTS = 8
SEQ = 2048
TOP_K = 2

TM_S = 256
TF = 1024
J = D_FF // TF
NT = (SEQ * TOP_K) // TM_S + N_EXPERTS
PAD = NT * TM_S


def _router_kernel(xb_ref, rw_ref, coef_ref):
    logits = jnp.dot(xb_ref[...], rw_ref[...],
                     preferred_element_type=jnp.float32)
    mx = jnp.max(logits, axis=-1, keepdims=True)
    ex = jnp.exp(logits - mx)
    p = ex / jnp.sum(ex, axis=-1, keepdims=True)
    lane = jax.lax.broadcasted_iota(jnp.int32, p.shape, 1)
    m1 = jnp.max(p, axis=-1, keepdims=True)
    i1 = jnp.min(jnp.where(p == m1, lane, N_EXPERTS), axis=-1, keepdims=True)
    mask1 = lane == i1
    pm = jnp.where(mask1, -1.0, p)
    m2 = jnp.max(pm, axis=-1, keepdims=True)
    i2 = jnp.min(jnp.where(pm == m2, lane, N_EXPERTS), axis=-1, keepdims=True)
    mask2 = lane == i2
    coef_ref[...] = jnp.where(mask1 | mask2, p, 0.0) / (m1 + m2)


def _moe_sparse_kernel(te_ref, tv_ref, x_ref, rt_ref, rwt_ref,
                       w1_ref, w2_ref, w3_ref, out_ref, gt_ref, xg_ref,
                       acc_ref):
    i = pl.program_id(0)
    j = pl.program_id(1)

    @pl.when((i == 0) & (j == 0))
    def _init():
        out_ref[...] = jnp.zeros_like(out_ref)

    @pl.when(tv_ref[i] == 1)
    def _work():

        @pl.when(j == 0)
        def _gather():
            rt = rt_ref[0]
            t_iota = jax.lax.broadcasted_iota(jnp.int32, (SEQ, TM_S), 0)
            gt = (t_iota == rt).astype(jnp.bfloat16)
            gt_ref[...] = gt
            xg_ref[...] = jax.lax.dot_general(
                gt, x_ref[...], (((0,), (0,)), ((), ())),
                preferred_element_type=jnp.float32).astype(jnp.bfloat16)
            acc_ref[...] = jnp.zeros_like(acc_ref)

        xg = xg_ref[...]
        gate = jnp.dot(xg, w1_ref[0], preferred_element_type=jnp.float32)
        val = jnp.dot(xg, w2_ref[0], preferred_element_type=jnp.float32)
        h = (gate * jax.nn.sigmoid(gate) * val).astype(jnp.bfloat16)
        acc_ref[...] += jnp.dot(h, w3_ref[0], preferred_element_type=jnp.float32)

        @pl.when(j == J - 1)
        def _scatter():
            w_row = rwt_ref[0].astype(jnp.bfloat16)
            gw = gt_ref[...] * w_row
            y = acc_ref[...].astype(jnp.bfloat16)
            out_ref[...] += jnp.dot(gw, y, preferred_element_type=jnp.float32)


def kernel(x, router_weight, w1, w2, w3):
    batch, seq, d = x.shape
    xb = x.reshape(seq, d).astype(jnp.bfloat16)
    rwb = router_weight.astype(jnp.bfloat16)
    w1b = w1.astype(jnp.bfloat16)
    w2b = w2.astype(jnp.bfloat16)
    w3b = w3.astype(jnp.bfloat16)

    coef = pl.pallas_call(
        _router_kernel,
        in_specs=[
            pl.BlockSpec((seq, d), lambda: (0, 0)),
            pl.BlockSpec((d, N_EXPERTS), lambda: (0, 0)),
        ],
        out_specs=pl.BlockSpec((seq, N_EXPERTS), lambda: (0, 0)),
        out_shape=jax.ShapeDtypeStruct((seq, N_EXPERTS), jnp.float32),
    )(xb, rwb)

    tkw, tki = jax.lax.top_k(coef, TOP_K)
    flat_e = tki.reshape(-1).astype(jnp.int32)
    flat_w = tkw.reshape(-1)
    order = jnp.argsort(flat_e, stable=True)
    sorted_e = flat_e[order]
    counts = jnp.sum(
        (flat_e[:, None] == jnp.arange(N_EXPERTS)[None, :]).astype(jnp.int32),
        axis=0)
    aligned = ((counts + TM_S - 1) // TM_S) * TM_S
    acum = jnp.cumsum(aligned)
    astart = acum - aligned
    gstart = jnp.cumsum(counts) - counts
    ranks = jnp.arange(seq * TOP_K, dtype=jnp.int32) - gstart[sorted_e]
    pos = astart[sorted_e] + ranks
    row_token = jnp.zeros((PAD,), jnp.int32).at[pos].set(
        (order // TOP_K).astype(jnp.int32))
    row_weight = jnp.zeros((PAD,), jnp.float32).at[pos].set(flat_w[order])
    tile_start = jnp.arange(NT, dtype=jnp.int32) * TM_S
    tile_expert = jnp.minimum(
        jnp.searchsorted(acum, tile_start, side="right").astype(jnp.int32),
        N_EXPERTS - 1)
    tile_valid = (tile_start < acum[-1]).astype(jnp.int32)

    out = pl.pallas_call(
        _moe_sparse_kernel,
        grid_spec=pltpu.PrefetchScalarGridSpec(
            num_scalar_prefetch=2,
            grid=(NT, J),
            in_specs=[
                pl.BlockSpec((seq, d), lambda i, j, te, tv: (0, 0)),
                pl.BlockSpec((1, 1, TM_S), lambda i, j, te, tv: (i, 0, 0)),
                pl.BlockSpec((1, 1, TM_S), lambda i, j, te, tv: (i, 0, 0)),
                pl.BlockSpec((1, D_MODEL, TF), lambda i, j, te, tv: (te[i], 0, j)),
                pl.BlockSpec((1, D_MODEL, TF), lambda i, j, te, tv: (te[i], 0, j)),
                pl.BlockSpec((1, TF, D_MODEL), lambda i, j, te, tv: (te[i], j, 0)),
            ],
            out_specs=pl.BlockSpec((seq, d), lambda i, j, te, tv: (0, 0)),
            scratch_shapes=[
                pltpu.VMEM((SEQ, TM_S), jnp.bfloat16),
                pltpu.VMEM((TM_S, D_MODEL), jnp.bfloat16),
                pltpu.VMEM((TM_S, D_MODEL), jnp.float32),
            ],
        ),
        out_shape=jax.ShapeDtypeStruct((seq, d), jnp.float32),
        compiler_params=pltpu.CompilerParams(
            dimension_semantics=("arbitrary", "arbitrary"),
        ),
    )(tile_expert, tile_valid, xb,
      row_token.reshape(NT, 1, TM_S), row_weight.reshape(NT, 1, TM_S),
      w1b, w2b, w3b)
    return out.reshape(batch, seq, d)


# speedup vs baseline: 1.3294x; 1.0403x over previous
import jax
import jax.numpy as jnp
from jax.experimental import pallas as pl
from jax.experimental.pallas import tpu as pltpu

D_MODEL = 1024
D_FF = 4096
N_EXPERTS = 8
SEQ = 2048
TOP_K = 2

TM_S = 512
TF = 1024
J = D_FF // TF
NT = (SEQ * TOP_K) // TM_S + N_EXPERTS
PAD = NT * TM_S


def _router_kernel(xb_ref, rw_ref, coef_ref):
    logits = jnp.dot(xb_ref[...], rw_ref[...],
                     preferred_element_type=jnp.float32)
    mx = jnp.max(logits, axis=-1, keepdims=True)
    ex = jnp.exp(logits - mx)
    p = ex / jnp.sum(ex, axis=-1, keepdims=True)
    lane = jax.lax.broadcasted_iota(jnp.int32, p.shape, 1)
    m1 = jnp.max(p, axis=-1, keepdims=True)
    i1 = jnp.min(jnp.where(p == m1, lane, N_EXPERTS), axis=-1, keepdims=True)
    mask1 = lane == i1
    pm = jnp.where(mask1, -1.0, p)
    m2 = jnp.max(pm, axis=-1, keepdims=True)
    i2 = jnp.min(jnp.where(pm == m2, lane, N_EXPERTS), axis=-1, keepdims=True)
    mask2 = lane == i2
    coef_ref[...] = jnp.where(mask1 | mask2, p, 0.0) / (m1 + m2)


def _moe_sparse_kernel(te_ref, tv_ref, x_ref, rt_ref, rwt_ref,
                       w1_ref, w2_ref, w3_ref, out_ref, gt_ref, xg_ref,
                       acc_ref):
    i = pl.program_id(0)
    j = pl.program_id(1)

    @pl.when((i == 0) & (j == 0))
    def _init():
        out_ref[...] = jnp.zeros_like(out_ref)

    @pl.when(tv_ref[i] == 1)
    def _work():

        @pl.when(j == 0)
        def _gather():
            rt = rt_ref[0]
            t_iota = jax.lax.broadcasted_iota(jnp.int32, (SEQ, TM_S), 0)
            gt = (t_iota == rt).astype(jnp.bfloat16)
            gt_ref[...] = gt
            xg_ref[...] = jax.lax.dot_general(
                gt, x_ref[...], (((0,), (0,)), ((), ())),
                preferred_element_type=jnp.float32).astype(jnp.bfloat16)
            acc_ref[...] = jnp.zeros_like(acc_ref)

        xg = xg_ref[...]
        gate = jnp.dot(xg, w1_ref[0], preferred_element_type=jnp.float32)
        val = jnp.dot(xg, w2_ref[0], preferred_element_type=jnp.float32)
        h = (gate * jax.nn.sigmoid(gate) * val).astype(jnp.bfloat16)
        acc_ref[...] += jnp.dot(h, w3_ref[0], preferred_element_type=jnp.float32)

        @pl.when(j == J - 1)
        def _scatter():
            w_row = rwt_ref[0].astype(jnp.bfloat16)
            gw = gt_ref[...] * w_row
            y = acc_ref[...].astype(jnp.bfloat16)
            out_ref[...] += jnp.dot(gw, y, preferred_element_type=jnp.float32)


def kernel(x, router_weight, w1, w2, w3):
    batch, seq, d = x.shape
    xb = x.reshape(seq, d).astype(jnp.bfloat16)
    rwb = router_weight.astype(jnp.bfloat16)
    w1b = w1.astype(jnp.bfloat16)
    w2b = w2.astype(jnp.bfloat16)
    w3b = w3.astype(jnp.bfloat16)

    coef = pl.pallas_call(
        _router_kernel,
        in_specs=[
            pl.BlockSpec((seq, d), lambda: (0, 0)),
            pl.BlockSpec((d, N_EXPERTS), lambda: (0, 0)),
        ],
        out_specs=pl.BlockSpec((seq, N_EXPERTS), lambda: (0, 0)),
        out_shape=jax.ShapeDtypeStruct((seq, N_EXPERTS), jnp.float32),
    )(xb, rwb)

    tkw, tki = jax.lax.top_k(coef, TOP_K)
    flat_e = tki.reshape(-1).astype(jnp.int32)
    flat_w = tkw.reshape(-1)
    order = jnp.argsort(flat_e, stable=True)
    sorted_e = flat_e[order]
    counts = jnp.sum(
        (flat_e[:, None] == jnp.arange(N_EXPERTS)[None, :]).astype(jnp.int32),
        axis=0)
    aligned = ((counts + TM_S - 1) // TM_S) * TM_S
    acum = jnp.cumsum(aligned)
    astart = acum - aligned
    gstart = jnp.cumsum(counts) - counts
    ranks = jnp.arange(seq * TOP_K, dtype=jnp.int32) - gstart[sorted_e]
    pos = astart[sorted_e] + ranks
    row_token = jnp.zeros((PAD,), jnp.int32).at[pos].set(
        (order // TOP_K).astype(jnp.int32))
    row_weight = jnp.zeros((PAD,), jnp.float32).at[pos].set(flat_w[order])
    tile_start = jnp.arange(NT, dtype=jnp.int32) * TM_S
    tile_expert = jnp.minimum(
        jnp.searchsorted(acum, tile_start, side="right").astype(jnp.int32),
        N_EXPERTS - 1)
    tile_valid = (tile_start < acum[-1]).astype(jnp.int32)

    out = pl.pallas_call(
        _moe_sparse_kernel,
        grid_spec=pltpu.PrefetchScalarGridSpec(
            num_scalar_prefetch=2,
            grid=(NT, J),
            in_specs=[
                pl.BlockSpec((seq, d), lambda i, j, te, tv: (0, 0)),
                pl.BlockSpec((1, 1, TM_S), lambda i, j, te, tv: (i, 0, 0)),
                pl.BlockSpec((1, 1, TM_S), lambda i, j, te, tv: (i, 0, 0)),
                pl.BlockSpec((1, D_MODEL, TF),
                             lambda i, j, te, tv:
                             (te[i], 0, jnp.where(tv[i] == 1, j, J - 1))),
                pl.BlockSpec((1, D_MODEL, TF),
                             lambda i, j, te, tv:
                             (te[i], 0, jnp.where(tv[i] == 1, j, J - 1))),
                pl.BlockSpec((1, TF, D_MODEL),
                             lambda i, j, te, tv:
                             (te[i], jnp.where(tv[i] == 1, j, J - 1), 0)),
            ],
            out_specs=pl.BlockSpec((seq, d), lambda i, j, te, tv: (0, 0)),
            scratch_shapes=[
                pltpu.VMEM((SEQ, TM_S), jnp.bfloat16),
                pltpu.VMEM((TM_S, D_MODEL), jnp.bfloat16),
                pltpu.VMEM((TM_S, D_MODEL), jnp.float32),
            ],
        ),
        out_shape=jax.ShapeDtypeStruct((seq, d), jnp.float32),
        compiler_params=pltpu.CompilerParams(
            dimension_semantics=("arbitrary", "arbitrary"),
        ),
    )(tile_expert, tile_valid, xb,
      row_token.reshape(NT, 1, TM_S), row_weight.reshape(NT, 1, TM_S),
      w1b, w2b, w3b)
    return out.reshape(batch, seq, d)
